# Initial kernel scaffold; baseline (speedup 1.0000x reference)
import functools

import jax
import jax.numpy as jnp
from jax import lax
from jax.experimental import pallas as pl
from jax.experimental.pallas import tpu as pltpu
from jax.experimental.pallas import tpu_sc as plsc

N_NODES_K = 100000
EDGES_K = 1600000
D = 64
DC = 16
NCHUNK = 4
NSUB = 16
ROWS_PER_TILE = N_NODES_K // NSUB
EDGES_PER_TILE = EDGES_K // NSUB
B = 2000
NBATCH = EDGES_PER_TILE // B
ZROWS = 1250


def _sc_layer_kernel(tbl, src_h, dst_h, val_h, out_h,
                     acc, rows, srcv, dstv, valv, zbuf, sem):
    c = lax.axis_index("c")
    s = lax.axis_index("s")

    @pl.loop(0, ZROWS)
    def _(i):
        zbuf[i] = jnp.zeros((16,), jnp.float32)

    def one_pass(tbl_k, out_k):
        r0 = s * ROWS_PER_TILE
        for j in range(ROWS_PER_TILE // ZROWS):
            pltpu.sync_copy(zbuf, acc.at[pl.ds(r0 + j * ZROWS, ZROWS)])
        plsc.subcore_barrier()

        e0 = s * EDGES_PER_TILE

        @pl.loop(0, NBATCH)
        def _(g):
            base = e0 + g * B
            pltpu.sync_copy(src_h.at[pl.ds(base, B)], srcv)
            pltpu.sync_copy(dst_h.at[pl.ds(base, B)], dstv)
            pltpu.sync_copy(val_h.at[pl.ds(base, B)], valv)
            pltpu.async_copy(tbl_k.at[srcv], rows, sem).wait()

            @pl.loop(0, B)
            def _(b):
                rows[b] = rows[b] * valv[b]

            pltpu.sync_copy(rows, acc.at[dstv], add=True)

        plsc.subcore_barrier()
        pltpu.sync_copy(acc.at[pl.ds(r0, ROWS_PER_TILE)],
                        out_k.at[pl.ds(r0, ROWS_PER_TILE)])

    for p in range(2):
        @pl.when(c == 0)
        def _(p=p):
            one_pass(tbl.at[p], out_h.at[p])

        @pl.when(c == 1)
        def _(p=p):
            one_pass(tbl.at[2 + p], out_h.at[2 + p])


def _sc_layer(e_chunked, src, dst, val):
    mesh = plsc.VectorSubcoreMesh(core_axis_name="c", subcore_axis_name="s")
    k = pl.kernel(
        _sc_layer_kernel,
        out_type=jax.ShapeDtypeStruct((NCHUNK, N_NODES_K, DC), jnp.float32),
        mesh=mesh,
        scratch_types=[
            pltpu.VMEM_SHARED((N_NODES_K, DC), jnp.float32),
            pltpu.VMEM((B, DC), jnp.float32),
            pltpu.VMEM((B,), jnp.int32),
            pltpu.VMEM((B,), jnp.int32),
            pltpu.VMEM((B,), jnp.float32),
            pltpu.VMEM((ZROWS, DC), jnp.float32),
            pltpu.SemaphoreType.DMA,
        ],
    )
    return k(e_chunked, src, dst, val)


def _tc_finish_kernel(base_ref, e1_ref, e2_ref, e3_ref, out_ref):
    def asm(e):
        return jnp.concatenate([e[0], e[1], e[2], e[3]], axis=1)

    out_ref[...] = (base_ref[...] + asm(e1_ref) + asm(e2_ref)
                    + asm(e3_ref)) * 0.25


def _tc_finish(base, e1, e2, e3, row_offset_blocks, n_rows):
    R = 2500
    grid = (n_rows // R,)
    espec = pl.BlockSpec((NCHUNK, R, DC),
                         lambda g: (0, g + row_offset_blocks, 0))
    return pl.pallas_call(
        _tc_finish_kernel,
        grid=grid,
        in_specs=[pl.BlockSpec((R, D), lambda g: (g, 0)),
                  espec, espec, espec],
        out_specs=pl.BlockSpec((R, D), lambda g: (g, 0)),
        out_shape=jax.ShapeDtypeStruct((n_rows, D), jnp.float32),
    )(base, e1, e2, e3)


@jax.jit
def kernel(adj_indices, adj_values, user_embeddings, item_embeddings):
    dst = adj_indices[0]
    src = adj_indices[1]
    e0 = jnp.concatenate([user_embeddings, item_embeddings], axis=0)
    e0c = e0.reshape(N_NODES_K, NCHUNK, DC).transpose(1, 0, 2)

    e1c = _sc_layer(e0c, src, dst, adj_values)
    e2c = _sc_layer(e1c, src, dst, adj_values)
    e3c = _sc_layer(e2c, src, dst, adj_values)

    n_users = user_embeddings.shape[0]
    user_final = _tc_finish(user_embeddings, e1c, e2c, e3c, 0, n_users)
    item_final = _tc_finish(item_embeddings, e1c, e2c, e3c,
                            n_users // 2500, item_embeddings.shape[0])
    return user_final, item_final


# speedup vs baseline: 6.6239x; 6.6239x over previous
import functools

import jax
import jax.numpy as jnp
from jax import lax
from jax.experimental import pallas as pl
from jax.experimental.pallas import tpu as pltpu
from jax.experimental.pallas import tpu_sc as plsc

N_NODES_K = 100000
N_PAD = 100096
EDGES_K = 1600000
D = 64
DC = 16
NCHUNK = 4
NSUB = 16
ROWS_PER_TILE = N_PAD // NSUB
EDGES_PER_TILE = EDGES_K // NSUB
B = 1000
NBATCH = EDGES_PER_TILE // B
ZROWS = 368


def _sc_layer_kernel(tbl, src_h, dst_h, val_h, out_h,
                     acc, rows, srcv, dstv, valv, sem):
    c = lax.axis_index("c")
    s = lax.axis_index("s")

    def one_pass(tbl_k, out_k):
        r0 = s * ROWS_PER_TILE
        @pl.loop(0, ZROWS)
        def _(i):
            rows[i] = jnp.zeros((16,), jnp.float32)

        for j in range(ROWS_PER_TILE // ZROWS):
            pltpu.sync_copy(rows.at[pl.ds(0, ZROWS)],
                            acc.at[pl.ds(r0 + j * ZROWS, ZROWS)])
        plsc.subcore_barrier()

        e0 = s * EDGES_PER_TILE

        @pl.loop(0, NBATCH)
        def _(g):
            base = e0 + g * B
            pltpu.sync_copy(src_h.at[pl.ds(base, B)], srcv)
            pltpu.sync_copy(dst_h.at[pl.ds(base, B)], dstv)
            pltpu.sync_copy(val_h.at[pl.ds(base, B)], valv)
            pltpu.async_copy(tbl_k.at[srcv], rows, sem).wait()

            @pl.loop(0, B, step=16)
            def _(b0):
                vv = valv[pl.ds(b0, 16)]
                for j in range(16):
                    rows[b0 + j] = rows[b0 + j] * vv[j]

            pltpu.sync_copy(rows, acc.at[dstv], add=True)

        plsc.subcore_barrier()
        pltpu.sync_copy(acc.at[pl.ds(r0, ROWS_PER_TILE)],
                        out_k.at[pl.ds(r0, ROWS_PER_TILE)])

    for p in range(2):
        @pl.when(c == 0)
        def _(p=p):
            one_pass(tbl.at[p], out_h.at[p])

        @pl.when(c == 1)
        def _(p=p):
            one_pass(tbl.at[2 + p], out_h.at[2 + p])


def _sc_layer(e_chunked, src, dst, val):
    mesh = plsc.VectorSubcoreMesh(core_axis_name="c", subcore_axis_name="s")
    k = pl.kernel(
        _sc_layer_kernel,
        out_type=jax.ShapeDtypeStruct((NCHUNK, N_PAD, DC), jnp.float32),
        mesh=mesh,
        scratch_types=[
            pltpu.VMEM_SHARED((N_PAD, DC), jnp.float32),
            pltpu.VMEM((B, DC), jnp.float32),
            pltpu.VMEM((B,), jnp.int32),
            pltpu.VMEM((B,), jnp.int32),
            pltpu.VMEM((B,), jnp.float32),
            pltpu.SemaphoreType.DMA,
        ],
        compiler_params=pltpu.CompilerParams(use_tc_tiling_on_sc=False),
    )
    return k(e_chunked, src, dst, val)


def _tc_finish_kernel(base_ref, e1_ref, e2_ref, e3_ref, out_ref):
    def asm(e):
        return jnp.concatenate([e[0], e[1], e[2], e[3]], axis=1)

    out_ref[...] = (base_ref[...] + asm(e1_ref) + asm(e2_ref)
                    + asm(e3_ref)) * 0.25


def _tc_finish(base, e1, e2, e3, row_offset_blocks, n_rows):
    R = 2000
    grid = (n_rows // R,)
    espec = pl.BlockSpec((NCHUNK, R, DC),
                         lambda g: (0, g + row_offset_blocks, 0))
    return pl.pallas_call(
        _tc_finish_kernel,
        grid=grid,
        in_specs=[pl.BlockSpec((R, D), lambda g: (g, 0)),
                  espec, espec, espec],
        out_specs=pl.BlockSpec((R, D), lambda g: (g, 0)),
        out_shape=jax.ShapeDtypeStruct((n_rows, D), jnp.float32),
    )(base, e1, e2, e3)


@jax.jit
def kernel(adj_indices, adj_values, user_embeddings, item_embeddings):
    dst = adj_indices[0]
    src = adj_indices[1]
    e0 = jnp.concatenate([user_embeddings, item_embeddings], axis=0)
    e0 = jnp.pad(e0, ((0, N_PAD - N_NODES_K), (0, 0)))
    e0c = e0.reshape(N_PAD, NCHUNK, DC).transpose(1, 0, 2)

    e1c = _sc_layer(e0c, src, dst, adj_values)
    e2c = _sc_layer(e1c, src, dst, adj_values)
    e3c = _sc_layer(e2c, src, dst, adj_values)

    n_users = user_embeddings.shape[0]
    user_final = _tc_finish(user_embeddings, e1c, e2c, e3c, 0, n_users)
    item_final = _tc_finish(item_embeddings, e1c, e2c, e3c,
                            n_users // 2000, item_embeddings.shape[0])
    return user_final, item_final


# speedup vs baseline: 9.7046x; 1.4651x over previous
import functools

import jax
import jax.numpy as jnp
from jax import lax
from jax.experimental import pallas as pl
from jax.experimental.pallas import tpu as pltpu
from jax.experimental.pallas import tpu_sc as plsc

N_NODES_K = 100000
N_PAD = 100096
EDGES_K = 1600000
D = 64
DC = 16
NCHUNK = 4
NSUB = 16
ROWS_PER_TILE = N_PAD // NSUB
EDGES_PER_TILE = EDGES_K // NSUB
B = 800
NBATCH = EDGES_PER_TILE // B
ZROWS = 368


def _sc_layer_kernel(tbl, ed_h, out_h, acc, rows2, ebuf,
                     gsem0, gsem1, ssem0, ssem1):
    c = lax.axis_index("c")
    s = lax.axis_index("s")
    gsem = (gsem0, gsem1)
    ssem = (ssem0, ssem1)

    def one_pass(tbl_k, out_k):
        r0 = s * ROWS_PER_TILE
        z = rows2.at[0]

        @pl.loop(0, ZROWS)
        def _(i):
            z[i] = jnp.zeros((16,), jnp.float32)

        for j in range(ROWS_PER_TILE // ZROWS):
            pltpu.sync_copy(z.at[pl.ds(0, ZROWS)],
                            acc.at[pl.ds(r0 + j * ZROWS, ZROWS)])
        plsc.subcore_barrier()

        e0 = s * EDGES_PER_TILE

        pltpu.sync_copy(ed_h.at[:, pl.ds(e0, B)], ebuf.at[0])
        pltpu.async_copy(tbl_k.at[ebuf.at[0, 0]], rows2.at[0], gsem[0])

        def step(g, sl):
            ot = 1 - sl

            @pl.when(g >= 1)
            def _():
                pltpu.make_async_copy(rows2.at[ot], acc.at[ebuf.at[ot, 1]],
                                      ssem[ot]).wait()

            @pl.when(g + 1 < NBATCH)
            def _():
                pltpu.sync_copy(ed_h.at[:, pl.ds(e0 + (g + 1) * B, B)],
                                ebuf.at[ot])
                pltpu.async_copy(tbl_k.at[ebuf.at[ot, 0]], rows2.at[ot],
                                 gsem[ot])

            pltpu.make_async_copy(tbl_k.at[ebuf.at[sl, 0]], rows2.at[sl],
                                  gsem[sl]).wait()
            rs = rows2.at[sl]

            @pl.loop(0, B, step=16)
            def _(b0):
                vv = plsc.bitcast(ebuf[sl, 2, pl.ds(b0, 16)], jnp.float32)
                for j in range(16):
                    rs[b0 + j] = rs[b0 + j] * vv[j]

            pltpu.async_copy(rows2.at[sl], acc.at[ebuf.at[sl, 1]], ssem[sl],
                             add=True)

        @pl.loop(0, NBATCH)
        def _(g):
            @pl.when(g % 2 == 0)
            def _():
                step(g, 0)

            @pl.when(g % 2 == 1)
            def _():
                step(g, 1)

        last = (NBATCH - 1) % 2
        pltpu.make_async_copy(rows2.at[last], acc.at[ebuf.at[last, 1]],
                              ssem[last]).wait()
        plsc.subcore_barrier()
        pltpu.sync_copy(acc.at[pl.ds(r0, ROWS_PER_TILE)],
                        out_k.at[pl.ds(r0, ROWS_PER_TILE)])

    for p in range(2):
        @pl.when(c == 0)
        def _(p=p):
            one_pass(tbl.at[p], out_h.at[p])

        @pl.when(c == 1)
        def _(p=p):
            one_pass(tbl.at[2 + p], out_h.at[2 + p])


def _sc_layer(e_chunked, edata):
    mesh = plsc.VectorSubcoreMesh(core_axis_name="c", subcore_axis_name="s")
    k = pl.kernel(
        _sc_layer_kernel,
        out_type=jax.ShapeDtypeStruct((NCHUNK, N_PAD, DC), jnp.float32),
        mesh=mesh,
        scratch_types=[
            pltpu.VMEM_SHARED((N_PAD, DC), jnp.float32),
            pltpu.VMEM((2, B, DC), jnp.float32),
            pltpu.VMEM((2, 3, B), jnp.int32),
            pltpu.SemaphoreType.DMA,
            pltpu.SemaphoreType.DMA,
            pltpu.SemaphoreType.DMA,
            pltpu.SemaphoreType.DMA,
        ],
        compiler_params=pltpu.CompilerParams(use_tc_tiling_on_sc=False,
                                            needs_layout_passes=False),
    )
    return k(e_chunked, edata)


def _tc_finish_kernel(base_ref, e1_ref, e2_ref, e3_ref, out_ref):
    def asm(e):
        return jnp.concatenate([e[0], e[1], e[2], e[3]], axis=1)

    out_ref[...] = (base_ref[...] + asm(e1_ref) + asm(e2_ref)
                    + asm(e3_ref)) * 0.25


def _tc_finish(base, e1, e2, e3, row_offset_blocks, n_rows):
    R = 2000
    grid = (n_rows // R,)
    espec = pl.BlockSpec((NCHUNK, R, DC),
                         lambda g: (0, g + row_offset_blocks, 0))
    return pl.pallas_call(
        _tc_finish_kernel,
        grid=grid,
        in_specs=[pl.BlockSpec((R, D), lambda g: (g, 0)),
                  espec, espec, espec],
        out_specs=pl.BlockSpec((R, D), lambda g: (g, 0)),
        out_shape=jax.ShapeDtypeStruct((n_rows, D), jnp.float32),
    )(base, e1, e2, e3)


@jax.jit
def kernel(adj_indices, adj_values, user_embeddings, item_embeddings):
    val_bits = jax.lax.bitcast_convert_type(adj_values, jnp.int32)
    edata = jnp.stack([adj_indices[1], adj_indices[0], val_bits])
    e0 = jnp.concatenate([user_embeddings, item_embeddings], axis=0)
    e0 = jnp.pad(e0, ((0, N_PAD - N_NODES_K), (0, 0)))
    e0c = e0.reshape(N_PAD, NCHUNK, DC).transpose(1, 0, 2)

    e1c = _sc_layer(e0c, edata)
    e2c = _sc_layer(e1c, edata)
    e3c = _sc_layer(e2c, edata)

    n_users = user_embeddings.shape[0]
    user_final = _tc_finish(user_embeddings, e1c, e2c, e3c, 0, n_users)
    item_final = _tc_finish(item_embeddings, e1c, e2c, e3c,
                            n_users // 2000, item_embeddings.shape[0])
    return user_final, item_final


# speedup vs baseline: 10.8628x; 1.1194x over previous
import jax
import jax.numpy as jnp
from jax import lax
from jax.experimental import pallas as pl
from jax.experimental.pallas import tpu as pltpu
from jax.experimental.pallas import tpu_sc as plsc

N_NODES_K = 100000
N_PAD = 100096
EDGES_K = 1600000
D = 64
DC = 16
NCHUNK = 4
NSUB = 16
ROWS_PER_TILE = N_PAD // NSUB
EDGES_PER_TILE = EDGES_K // NSUB
B = 400
H = B // 2
NBATCH = EDGES_PER_TILE // B
NSLOT = 4
ZROWS = 368


def _sc_layer_kernel(tbl, ed_h, out_h, acc, rows4, ebuf, *sems):
    c = lax.axis_index("c")
    s = lax.axis_index("s")
    gsem = sems[0:4]
    hsem = sems[4:8]
    ssem = sems[8:12]
    isem = sems[12:16]

    def start_gather(tbl_k, j):
        pltpu.async_copy(tbl_k.at[ebuf.at[j, 0].at[pl.ds(0, H)]],
                         rows4.at[j].at[pl.ds(0, H)], gsem[j])
        pltpu.async_copy(tbl_k.at[ebuf.at[j, 0].at[pl.ds(H, H)]],
                         rows4.at[j].at[pl.ds(H, H)], hsem[j])

    def wait_gather(tbl_k, j):
        pltpu.make_async_copy(tbl_k.at[ebuf.at[j, 0].at[pl.ds(0, H)]],
                              rows4.at[j].at[pl.ds(0, H)], gsem[j]).wait()
        pltpu.make_async_copy(tbl_k.at[ebuf.at[j, 0].at[pl.ds(H, H)]],
                              rows4.at[j].at[pl.ds(H, H)], hsem[j]).wait()

    def wait_scatter(j):
        pltpu.make_async_copy(rows4.at[j], acc.at[ebuf.at[j, 1]],
                              ssem[j]).wait()

    def one_pass(tbl_k, out_k):
        r0 = s * ROWS_PER_TILE
        z = rows4.at[0]

        @pl.loop(0, ZROWS)
        def _(i):
            z[i] = jnp.zeros((16,), jnp.float32)

        for j in range(ROWS_PER_TILE // ZROWS):
            pltpu.sync_copy(z.at[pl.ds(0, ZROWS)],
                            acc.at[pl.ds(r0 + j * ZROWS, ZROWS)])
        plsc.subcore_barrier()

        e0 = s * EDGES_PER_TILE

        pltpu.sync_copy(ed_h.at[:, pl.ds(e0, B)], ebuf.at[0])
        pltpu.sync_copy(ed_h.at[:, pl.ds(e0 + B, B)], ebuf.at[1])
        start_gather(tbl_k, 0)

        def step(g, sl):
            j1 = (sl + 1) % NSLOT
            j2 = (sl + 2) % NSLOT

            @pl.when(g + 1 < NBATCH)
            def _():
                @pl.when(g + 1 >= 2)
                def _():
                    pltpu.make_async_copy(ed_h.at[:, pl.ds(e0, B)],
                                          ebuf.at[j1], isem[j1]).wait()
                start_gather(tbl_k, j1)

            @pl.when(g >= 2)
            def _():
                wait_scatter(j2)

            @pl.when(g + 2 < NBATCH)
            def _():
                pltpu.async_copy(ed_h.at[:, pl.ds(e0 + (g + 2) * B, B)],
                                 ebuf.at[j2], isem[j2])

            wait_gather(tbl_k, sl)
            rs = rows4.at[sl]

            @pl.loop(0, B, step=16)
            def _(b0):
                vv = plsc.bitcast(ebuf[sl, 2, pl.ds(b0, 16)], jnp.float32)
                for j in range(16):
                    rs[b0 + j] = rs[b0 + j] * vv[j]

            pltpu.async_copy(rows4.at[sl], acc.at[ebuf.at[sl, 1]], ssem[sl],
                             add=True)

        @pl.loop(0, NBATCH)
        def _(g):
            for k in range(NSLOT):
                @pl.when(g % NSLOT == k)
                def _(k=k):
                    step(g, k)

        wait_scatter((NBATCH - 2) % NSLOT)
        wait_scatter((NBATCH - 1) % NSLOT)
        plsc.subcore_barrier()
        pltpu.sync_copy(acc.at[pl.ds(r0, ROWS_PER_TILE)],
                        out_k.at[pl.ds(r0, ROWS_PER_TILE)])

    for p in range(2):
        @pl.when(c == 0)
        def _(p=p):
            one_pass(tbl.at[p], out_h.at[p])

        @pl.when(c == 1)
        def _(p=p):
            one_pass(tbl.at[2 + p], out_h.at[2 + p])


def _sc_layer(e_chunked, edata):
    mesh = plsc.VectorSubcoreMesh(core_axis_name="c", subcore_axis_name="s")
    k = pl.kernel(
        _sc_layer_kernel,
        out_type=jax.ShapeDtypeStruct((NCHUNK, N_PAD, DC), jnp.float32),
        mesh=mesh,
        scratch_types=[
            pltpu.VMEM_SHARED((N_PAD, DC), jnp.float32),
            pltpu.VMEM((NSLOT, B, DC), jnp.float32),
            pltpu.VMEM((NSLOT, 3, B), jnp.int32),
        ] + [pltpu.SemaphoreType.DMA] * 16,
        compiler_params=pltpu.CompilerParams(use_tc_tiling_on_sc=False,
                                            needs_layout_passes=False),
    )
    return k(e_chunked, edata)


def _tc_finish_kernel(base_ref, e1_ref, e2_ref, e3_ref, out_ref):
    def asm(e):
        return jnp.concatenate([e[0], e[1], e[2], e[3]], axis=1)

    out_ref[...] = (base_ref[...] + asm(e1_ref) + asm(e2_ref)
                    + asm(e3_ref)) * 0.25


def _tc_finish(base, e1, e2, e3, row_offset_blocks, n_rows):
    R = 2000
    grid = (n_rows // R,)
    espec = pl.BlockSpec((NCHUNK, R, DC),
                         lambda g: (0, g + row_offset_blocks, 0))
    return pl.pallas_call(
        _tc_finish_kernel,
        grid=grid,
        in_specs=[pl.BlockSpec((R, D), lambda g: (g, 0)),
                  espec, espec, espec],
        out_specs=pl.BlockSpec((R, D), lambda g: (g, 0)),
        out_shape=jax.ShapeDtypeStruct((n_rows, D), jnp.float32),
    )(base, e1, e2, e3)


@jax.jit
def kernel(adj_indices, adj_values, user_embeddings, item_embeddings):
    val_bits = jax.lax.bitcast_convert_type(adj_values, jnp.int32)
    edata = jnp.stack([adj_indices[1], adj_indices[0], val_bits])
    e0 = jnp.concatenate([user_embeddings, item_embeddings], axis=0)
    e0 = jnp.pad(e0, ((0, N_PAD - N_NODES_K), (0, 0)))
    e0c = e0.reshape(N_PAD, NCHUNK, DC).transpose(1, 0, 2)

    e1c = _sc_layer(e0c, edata)
    e2c = _sc_layer(e1c, edata)
    e3c = _sc_layer(e2c, edata)

    n_users = user_embeddings.shape[0]
    user_final = _tc_finish(user_embeddings, e1c, e2c, e3c, 0, n_users)
    item_final = _tc_finish(item_embeddings, e1c, e2c, e3c,
                            n_users // 2000, item_embeddings.shape[0])
    return user_final, item_final
